# baseline (speedup 1.0000x reference)
import functools

import jax
import jax.numpy as jnp
from jax import lax
from jax.experimental import pallas as pl
from jax.experimental.pallas import tpu as pltpu
from jax.experimental.pallas import tpu_sc as plsc

_NC = 2
_NS = 16
_NW = _NC * _NS

_SC_BATCHES = 4


def _argmin_body(x_ref, w_ref, idx_ref):
    w = w_ref[...]
    xb = x_ref[0]
    x_sq = jnp.sum(xb * xb, axis=0)[None, :]
    e_sq = jnp.sum(w * w, axis=0)[:, None]
    mm2 = lax.dot_general(w * -2.0, xb, (((0,), (0,)), ((), ())))
    dist = (x_sq + e_sq) + mm2
    idx_ref[0, 0, :] = jnp.argmin(dist, axis=0).astype(jnp.int32)


def _argmin_call(x3, W):
    B, D, HW = x3.shape
    K = W.shape[1]
    return pl.pallas_call(
        _argmin_body,
        grid=(B,),
        in_specs=[
            pl.BlockSpec((1, D, HW), lambda b: (b, 0, 0)),
            pl.BlockSpec((D, K), lambda b: (0, 0)),
        ],
        out_specs=pl.BlockSpec((1, 1, HW), lambda b: (b, 0, 0)),
        out_shape=jax.ShapeDtypeStruct((B, 1, HW), jnp.int32),
    )(x3, W)


def _argmin_onehot_body(x_ref, w_ref, idx_ref, res_ref):
    w = w_ref[...]
    xb = x_ref[0]
    x_sq = jnp.sum(xb * xb, axis=0)[None, :]
    e_sq = jnp.sum(w * w, axis=0)[:, None]
    mm2 = lax.dot_general(w * -2.0, xb, (((0,), (0,)), ((), ())))
    dist = (x_sq + e_sq) + mm2
    am = jnp.argmin(dist, axis=0)
    idx_ref[0, 0, :] = am.astype(jnp.int32)
    K, HW = dist.shape
    iota = lax.broadcasted_iota(jnp.int32, (K, HW), 0)
    oh = (iota == am[None, :]).astype(jnp.float32)
    res_ref[0] = lax.dot_general(w, oh, (((1,), (0,)), ((), ())),
                                 precision=lax.Precision.HIGHEST)


def _argmin_onehot_call(x3, W):
    Bc, D, HW = x3.shape
    K = W.shape[1]
    return pl.pallas_call(
        _argmin_onehot_body,
        grid=(Bc,),
        in_specs=[
            pl.BlockSpec((1, D, HW), lambda b: (b, 0, 0)),
            pl.BlockSpec((D, K), lambda b: (0, 0)),
        ],
        out_specs=[
            pl.BlockSpec((1, 1, HW), lambda b: (b, 0, 0)),
            pl.BlockSpec((1, D, HW), lambda b: (b, 0, 0)),
        ],
        out_shape=[
            jax.ShapeDtypeStruct((Bc, 1, HW), jnp.int32),
            jax.ShapeDtypeStruct((Bc, D, HW), jnp.float32),
        ],
    )(x3, W)


def _gather_call(WT, idx_flat):
    K, D = WT.shape
    N = idx_flat.shape[0]
    bpw = N // _NW
    mesh = plsc.VectorSubcoreMesh(core_axis_name="c", subcore_axis_name="s")

    @functools.partial(
        pl.kernel,
        mesh=mesh,
        out_type=jax.ShapeDtypeStruct((N, D), jnp.float32),
        scratch_types=[
            pltpu.VMEM((bpw,), jnp.int32),
            pltpu.VMEM((bpw, D), jnp.float32),
            pltpu.SemaphoreType.DMA,
            pltpu.SemaphoreType.DMA,
        ],
    )
    def gather(table_hbm, idx_hbm, out_hbm, idx_v, rows_v, isem, osem):
        wid = lax.axis_index("s") * _NC + lax.axis_index("c")
        base = wid * bpw
        pltpu.sync_copy(idx_hbm.at[pl.ds(base, bpw)], idx_v)
        pltpu.async_copy(table_hbm.at[idx_v], rows_v, isem).wait()
        pltpu.async_copy(rows_v, out_hbm.at[pl.ds(base, bpw)], osem).wait()

    return gather(WT, idx_flat)


def kernel(x, W):
    B, D, H, Wd = x.shape
    HW = H * Wd
    x3 = x.reshape(B, D, HW)
    S = _SC_BATCHES
    idxA = _argmin_call(x3[:S], W)
    gatheredA = _gather_call(W.T, idxA.reshape(S * HW))
    dep = jnp.minimum(idxA[0, 0, 0], 0).astype(jnp.float32)
    idxB, resB = _argmin_onehot_call(x3[S:], W + dep)
    resA = gatheredA.reshape(S, H, Wd, D).transpose(0, 3, 1, 2)
    result = jnp.concatenate(
        [resA, resB.reshape(B - S, D, H, Wd)], axis=0)
    argmin_out = jnp.concatenate(
        [idxA, idxB], axis=0).reshape(B, H, Wd)
    return result, argmin_out


# speedup vs baseline: 1.4064x; 1.4064x over previous
import functools

import jax
import jax.numpy as jnp
from jax import lax
from jax.experimental import pallas as pl
from jax.experimental.pallas import tpu as pltpu
from jax.experimental.pallas import tpu_sc as plsc

_NC = 2
_NS = 16
_NW = _NC * _NS


def _argmin_body(x_ref, w_ref, idx_ref):
    w = w_ref[...]
    xb = x_ref[0]
    x_sq = jnp.sum(xb * xb, axis=0)[None, :]
    e_sq = jnp.sum(w * w, axis=0)[:, None]
    mm2 = lax.dot_general(w * -2.0, xb, (((0,), (0,)), ((), ())))
    dist = (x_sq + e_sq) + mm2
    idx_ref[0, 0, :] = jnp.argmin(dist, axis=0).astype(jnp.int32)


def _argmin_call(x3, W):
    B, D, HW = x3.shape
    K = W.shape[1]
    return pl.pallas_call(
        _argmin_body,
        grid=(B,),
        in_specs=[
            pl.BlockSpec((1, D, HW), lambda b: (b, 0, 0)),
            pl.BlockSpec((D, K), lambda b: (0, 0)),
        ],
        out_specs=pl.BlockSpec((1, 1, HW), lambda b: (b, 0, 0)),
        out_shape=jax.ShapeDtypeStruct((B, 1, HW), jnp.int32),
    )(x3, W)


def _gather_call(WT, idx_flat):
    K, D = WT.shape
    N = idx_flat.shape[0]
    bpw = N // _NW
    mesh = plsc.VectorSubcoreMesh(core_axis_name="c", subcore_axis_name="s")

    @functools.partial(
        pl.kernel,
        mesh=mesh,
        out_type=jax.ShapeDtypeStruct((N, D), jnp.float32),
        scratch_types=[
            pltpu.VMEM((bpw,), jnp.int32),
            pltpu.VMEM((bpw, D), jnp.float32),
            pltpu.SemaphoreType.DMA,
            pltpu.SemaphoreType.DMA,
        ],
    )
    def gather(table_hbm, idx_hbm, out_hbm, idx_v, rows_v, isem, osem):
        wid = lax.axis_index("s") * _NC + lax.axis_index("c")
        base = wid * bpw
        pltpu.sync_copy(idx_hbm.at[pl.ds(base, bpw)], idx_v)
        pltpu.async_copy(table_hbm.at[idx_v], rows_v, isem).wait()
        pltpu.async_copy(rows_v, out_hbm.at[pl.ds(base, bpw)], osem).wait()

    return gather(WT, idx_flat)


def kernel(x, W):
    B, D, H, Wd = x.shape
    HW = H * Wd
    x3 = x.reshape(B, D, HW)
    idx3 = _argmin_call(x3, W)
    idx_flat = idx3.reshape(B * HW)
    gathered = _gather_call(W.T, idx_flat)
    result = gathered.reshape(B, H, Wd, D).transpose(0, 3, 1, 2)
    argmin_out = idx3.reshape(B, H, Wd)
    return result, argmin_out
